# baseline (speedup 1.0000x reference)
import functools

import jax
import jax.numpy as jnp
from jax import lax
from jax.experimental import pallas as pl
from jax.experimental.pallas import tpu as pltpu
from jax.experimental.pallas import tpu_sc as plsc

NC = 2
NS = 16
LANES = 128
FRAC0 = 0.46


def _sc_aggregate(nacc, e, ta, tb, x, edges, zacc, zcnt, ones):
    n, d = x.shape
    rows_per_tile = nacc // NS
    q0, r0 = divmod(ta, NS)
    q1, r1 = divmod(tb, NS)
    assert q0 >= 2 and q1 >= 2
    mesh = plsc.VectorSubcoreMesh(core_axis_name="c", subcore_axis_name="s")

    @functools.partial(
        pl.kernel,
        out_type=(
            jax.ShapeDtypeStruct((NC, nacc, d), jnp.float32),
            jax.ShapeDtypeStruct((NC * nacc,), jnp.float32),
        ),
        mesh=mesh,
        scratch_types=[
            pltpu.VMEM((2, LANES), jnp.int32),
            pltpu.VMEM((2, LANES), jnp.int32),
            pltpu.VMEM((2, LANES, d), jnp.float32),
            pltpu.VMEM((LANES,), jnp.float32),
            pltpu.VMEM_SHARED((nacc, d), jnp.float32),
            pltpu.VMEM_SHARED((nacc,), jnp.float32),
            pltpu.SemaphoreType.DMA,
            pltpu.SemaphoreType.DMA,
        ],
    )
    def agg(x_hbm, edges_hbm, zacc_hbm, zcnt_hbm, ones_hbm,
            sum_out, cnt_out, idxc, idxr, gbuf, onesbuf, acc_sh, cnt_sh,
            sem0, sem1):
        c = lax.axis_index("c")
        s = lax.axis_index("s")
        start_chunk = lax.select(
            c == 0,
            s * q0 + jnp.minimum(s, r0),
            ta + s * q1 + jnp.minimum(s, r1))
        mychunks = lax.select(
            c == 0,
            q0 + jnp.where(s < r0, 1, 0),
            q1 + jnp.where(s < r1, 1, 0))
        r_0 = s * rows_per_tile
        sems = (sem0, sem1)
        pltpu.sync_copy(zacc_hbm.at[pl.ds(r_0, rows_per_tile)],
                        acc_sh.at[pl.ds(r_0, rows_per_tile)])
        pltpu.sync_copy(zcnt_hbm.at[pl.ds(r_0, rows_per_tile)],
                        cnt_sh.at[pl.ds(r_0, rows_per_tile)])
        pltpu.sync_copy(ones_hbm, onesbuf)
        plsc.subcore_barrier()

        def issue(k, i):
            base = (start_chunk + i) * LANES
            pltpu.sync_copy(edges_hbm.at[pl.ds(e + base, LANES)], idxc.at[k])
            pltpu.sync_copy(edges_hbm.at[pl.ds(base, LANES)], idxr.at[k])
            pltpu.async_copy(x_hbm.at[idxc.at[k]], gbuf.at[k], sems[k])

        def process(k, i):
            pltpu.make_async_copy(x_hbm.at[pl.ds(0, LANES)], gbuf.at[k],
                                  sems[k]).wait()
            pltpu.sync_copy(gbuf.at[k], acc_sh.at[idxr.at[k]], add=True)
            pltpu.sync_copy(onesbuf, cnt_sh.at[idxr.at[k]], add=True)

            @pl.when(i + 2 < mychunks)
            def _():
                issue(k, i + 2)

        issue(0, 0)
        issue(1, 1)

        def step(i, carry):
            @pl.when(i % 2 == 0)
            def _():
                process(0, i)

            @pl.when(i % 2 != 0)
            def _():
                process(1, i)

            return carry

        lax.fori_loop(0, mychunks, step, 0)
        plsc.subcore_barrier()
        pltpu.sync_copy(acc_sh.at[pl.ds(r_0, rows_per_tile)],
                        sum_out.at[c, pl.ds(r_0, rows_per_tile)])
        pltpu.sync_copy(cnt_sh.at[pl.ds(r_0, rows_per_tile)],
                        cnt_out.at[pl.ds(c * nacc + r_0, rows_per_tile)])

    return agg(x, edges, zacc, zcnt, ones)


def _tc_body(x_ref, sums_ref, cnt0_ref, cnt1_ref, wt_ref, b_ref, o_ref):
    s = sums_ref[0] + sums_ref[1]
    cnt = cnt0_ref[...] + cnt1_ref[...]
    mean = s / (cnt + 1e-9)
    d = x_ref.shape[1]
    out = (jnp.dot(x_ref[...], wt_ref[pl.ds(0, d)],
                   preferred_element_type=jnp.float32)
           + jnp.dot(mean, wt_ref[pl.ds(d, d)],
                     preferred_element_type=jnp.float32)
           + b_ref[...])
    o_ref[...] = jnp.maximum(out, 0.0)


def kernel(x, edge_index, W, b):
    n, d = x.shape
    e = edge_index.shape[1]
    nacc = ((n + 1 + NS * 16 - 1) // (NS * 16)) * (NS * 16)

    if e % LANES:
        pad_len = LANES - e % LANES
        row = jnp.concatenate([edge_index[0],
                               jnp.full((pad_len,), n, jnp.int32)])
        col = jnp.concatenate([edge_index[1],
                               jnp.zeros((pad_len,), jnp.int32)])
        edges = jnp.concatenate([row, col])
        e_pad = e + pad_len
    else:
        edges = edge_index.reshape(2 * e)
        e_pad = e
    t = e_pad // LANES
    ta = min(t - 2 * NS, max(2 * NS, round(t * FRAC0)))
    tb = t - ta

    zacc = jnp.zeros((nacc, d), jnp.float32)
    zcnt = jnp.zeros((nacc,), jnp.float32)
    ones = jnp.ones((LANES,), jnp.float32)

    sums, cnt_flat = _sc_aggregate(nacc, e_pad, ta, tb, x, edges, zacc,
                                   zcnt, ones)
    cnt_col = cnt_flat.reshape(NC * nacc, 1)

    wt = W.T
    d_out = W.shape[0]
    block_rows = 1024
    grid = (n + block_rows - 1) // block_rows
    nblk = nacc // block_rows
    out = pl.pallas_call(
        _tc_body,
        grid=(grid,),
        in_specs=[
            pl.BlockSpec((block_rows, d), lambda i: (i, 0)),
            pl.BlockSpec((NC, block_rows, d), lambda i: (0, i, 0)),
            pl.BlockSpec((block_rows, 1), lambda i: (i, 0)),
            pl.BlockSpec((block_rows, 1), lambda i: (nblk + i, 0)),
            pl.BlockSpec((2 * d, d_out), lambda i: (0, 0)),
            pl.BlockSpec((1, d_out), lambda i: (0, 0)),
        ],
        out_specs=pl.BlockSpec((block_rows, d_out), lambda i: (i, 0)),
        out_shape=jax.ShapeDtypeStruct((n, d_out), jnp.float32),
    )(x, sums, cnt_col, cnt_col, wt, b.reshape(1, d_out))
    return out


# speedup vs baseline: 1.0430x; 1.0430x over previous
import functools

import jax
import jax.numpy as jnp
from jax import lax
from jax.experimental import pallas as pl
from jax.experimental.pallas import tpu as pltpu
from jax.experimental.pallas import tpu_sc as plsc

NC = 2
NS = 16
LANES = 128
FRAC0 = 0.50


def _sc_aggregate(nacc, e, ta, tb, x, edges, zacc, zcnt, ones):
    n, d = x.shape
    rows_per_tile = nacc // NS
    q0, r0 = divmod(ta, NS)
    q1, r1 = divmod(tb, NS)
    assert q0 >= 2 and q1 >= 2
    mesh = plsc.VectorSubcoreMesh(core_axis_name="c", subcore_axis_name="s")

    @functools.partial(
        pl.kernel,
        out_type=(
            jax.ShapeDtypeStruct((NC, nacc, d), jnp.float32),
            jax.ShapeDtypeStruct((NC * nacc,), jnp.float32),
        ),
        mesh=mesh,
        scratch_types=[
            pltpu.VMEM((2, LANES), jnp.int32),
            pltpu.VMEM((2, LANES), jnp.int32),
            pltpu.VMEM((2, LANES, d), jnp.float32),
            pltpu.VMEM((LANES,), jnp.float32),
            pltpu.VMEM_SHARED((nacc, d), jnp.float32),
            pltpu.VMEM_SHARED((nacc,), jnp.float32),
            pltpu.SemaphoreType.DMA,
            pltpu.SemaphoreType.DMA,
        ],
    )
    def agg(x_hbm, edges_hbm, zacc_hbm, zcnt_hbm, ones_hbm,
            sum_out, cnt_out, idxc, idxr, gbuf, onesbuf, acc_sh, cnt_sh,
            sem0, sem1):
        c = lax.axis_index("c")
        s = lax.axis_index("s")
        start_chunk = lax.select(
            c == 0,
            s * q0 + jnp.minimum(s, r0),
            ta + s * q1 + jnp.minimum(s, r1))
        mychunks = lax.select(
            c == 0,
            q0 + jnp.where(s < r0, 1, 0),
            q1 + jnp.where(s < r1, 1, 0))
        r_0 = s * rows_per_tile
        sems = (sem0, sem1)
        pltpu.sync_copy(zacc_hbm.at[pl.ds(r_0, rows_per_tile)],
                        acc_sh.at[pl.ds(r_0, rows_per_tile)])
        pltpu.sync_copy(zcnt_hbm.at[pl.ds(r_0, rows_per_tile)],
                        cnt_sh.at[pl.ds(r_0, rows_per_tile)])
        pltpu.sync_copy(ones_hbm, onesbuf)
        plsc.subcore_barrier()

        def issue(k, i):
            base = (start_chunk + i) * LANES
            pltpu.sync_copy(edges_hbm.at[pl.ds(e + base, LANES)], idxc.at[k])
            pltpu.sync_copy(edges_hbm.at[pl.ds(base, LANES)], idxr.at[k])
            pltpu.async_copy(x_hbm.at[idxc.at[k]], gbuf.at[k], sems[k])

        def process(k, i):
            pltpu.make_async_copy(x_hbm.at[pl.ds(0, LANES)], gbuf.at[k],
                                  sems[k]).wait()
            pltpu.sync_copy(gbuf.at[k], acc_sh.at[idxr.at[k]], add=True)
            pltpu.sync_copy(onesbuf, cnt_sh.at[idxr.at[k]], add=True)

            @pl.when(i + 2 < mychunks)
            def _():
                issue(k, i + 2)

        issue(0, 0)
        issue(1, 1)

        def step(i, carry):
            @pl.when(i % 2 == 0)
            def _():
                process(0, i)

            @pl.when(i % 2 != 0)
            def _():
                process(1, i)

            return carry

        lax.fori_loop(0, mychunks, step, 0)
        plsc.subcore_barrier()
        pltpu.sync_copy(acc_sh.at[pl.ds(r_0, rows_per_tile)],
                        sum_out.at[c, pl.ds(r_0, rows_per_tile)])
        pltpu.sync_copy(cnt_sh.at[pl.ds(r_0, rows_per_tile)],
                        cnt_out.at[pl.ds(c * nacc + r_0, rows_per_tile)])

    return agg(x, edges, zacc, zcnt, ones)


def _tc_body(x_ref, sums_ref, cnt0_ref, cnt1_ref, wt_ref, b_ref, o_ref):
    s = sums_ref[0] + sums_ref[1]
    cnt = cnt0_ref[...] + cnt1_ref[...]
    mean = s / (cnt + 1e-9)
    d = x_ref.shape[1]
    out = (jnp.dot(x_ref[...], wt_ref[pl.ds(0, d)],
                   preferred_element_type=jnp.float32)
           + jnp.dot(mean, wt_ref[pl.ds(d, d)],
                     preferred_element_type=jnp.float32)
           + b_ref[...])
    o_ref[...] = jnp.maximum(out, 0.0)


def kernel(x, edge_index, W, b):
    n, d = x.shape
    e = edge_index.shape[1]
    nacc = ((n + 1 + NS * 16 - 1) // (NS * 16)) * (NS * 16)

    if e % LANES:
        pad_len = LANES - e % LANES
        row = jnp.concatenate([edge_index[0],
                               jnp.full((pad_len,), n, jnp.int32)])
        col = jnp.concatenate([edge_index[1],
                               jnp.zeros((pad_len,), jnp.int32)])
        edges = jnp.concatenate([row, col])
        e_pad = e + pad_len
    else:
        edges = edge_index.reshape(2 * e)
        e_pad = e
    t = e_pad // LANES
    ta = min(t - 2 * NS, max(2 * NS, round(t * FRAC0)))
    tb = t - ta

    zacc = jnp.zeros((nacc, d), jnp.float32)
    zcnt = jnp.zeros((nacc,), jnp.float32)
    ones = jnp.ones((LANES,), jnp.float32)

    sums, cnt_flat = _sc_aggregate(nacc, e_pad, ta, tb, x, edges, zacc,
                                   zcnt, ones)
    cnt_col = cnt_flat.reshape(NC * nacc, 1)

    wt = W.T
    d_out = W.shape[0]
    block_rows = 1024
    grid = (n + block_rows - 1) // block_rows
    nblk = nacc // block_rows
    out = pl.pallas_call(
        _tc_body,
        grid=(grid,),
        in_specs=[
            pl.BlockSpec((block_rows, d), lambda i: (i, 0)),
            pl.BlockSpec((NC, block_rows, d), lambda i: (0, i, 0)),
            pl.BlockSpec((block_rows, 1), lambda i: (i, 0)),
            pl.BlockSpec((block_rows, 1), lambda i: (nblk + i, 0)),
            pl.BlockSpec((2 * d, d_out), lambda i: (0, 0)),
            pl.BlockSpec((1, d_out), lambda i: (0, 0)),
        ],
        out_specs=pl.BlockSpec((block_rows, d_out), lambda i: (i, 0)),
        out_shape=jax.ShapeDtypeStruct((n, d_out), jnp.float32),
    )(x, sums, cnt_col, cnt_col, wt, b.reshape(1, d_out))
    return out
